# Initial kernel scaffold; baseline (speedup 1.0000x reference)
import jax
import jax.numpy as jnp
from jax.experimental import pallas as pl


def kernel(x, edge_index, edge_attr, batch, params):
    raise NotImplementedError("write your pallas kernel here")


# speedup vs baseline: 1.2442x; 1.2442x over previous
import functools

import jax
import jax.numpy as jnp
from jax import lax
from jax.experimental import pallas as pl
from jax.experimental.pallas import tpu as pltpu
from jax.experimental.pallas import tpu_sc as plsc

N = 10000
E = 320000
DF = 128
H = 128
ED = 16
OUT = 6
NG = 64

NC = 2
NS = 16
NW = NC * NS
K = 80
NPW = 320
N_PAD = NW * NPW
PADE = K

_sc_mesh = plsc.VectorSubcoreMesh(core_axis_name="c", subcore_axis_name="s")

_OFFS = [10080 * t for t in range(1, 12)] + [110880 + 9840 * t for t in range(1, 5)]
_P_LIST = sorted([b + o for b in (0, E // 2) for o in _OFFS] + [E // 2])
_NSEG = len(_P_LIST) + 1
_SEGW = 48


@functools.partial(
    pl.kernel,
    mesh=_sc_mesh,
    out_type=jax.ShapeDtypeStruct((N_PAD, H), jnp.float32),
    scratch_types=[
        pltpu.VMEM((_SEGW,), jnp.int32),
        pltpu.VMEM((K,), jnp.int32),
        pltpu.VMEM((K,), jnp.int32),
        pltpu.VMEM((K + 16,), jnp.int32),
        pltpu.VMEM((K, H), jnp.float32),
        pltpu.VMEM((K, H), jnp.float32),
        pltpu.VMEM((NPW, H), jnp.float32),
        pltpu.VMEM((NPW, H), jnp.float32),
        pltpu.SemaphoreType.DMA,
        pltpu.SemaphoreType.DMA,
    ],
)
def _sc_aggregate(h_hbm, ea_hbm, ids_hbm, srcs_hbm, dsts_hbm, segs_hbm, out_hbm,
                  segs_v, ids_v, src_v, dst_v, msg_v, ea_v, acc_v, part_v,
                  sem1, sem2):
    c = lax.axis_index("c")
    s = lax.axis_index("s")
    w = c * NS + s
    base = w * NPW

    pltpu.sync_copy(segs_hbm.at[w], segs_v)

    def _zero(ref):
        def _zrow(r, carry):
            for j in range(H // 16):
                ref[r, pl.ds(j * 16, 16)] = jnp.zeros((16,), jnp.float32)
            return carry
        lax.fori_loop(0, NPW, _zrow, 0)

    _zero(acc_v)
    _zero(part_v)

    def _segment(ks, carry):
        sv = segs_v[pl.ds(ks, 16)]
        b0 = sv[0]
        b1 = sv[1]

        @pl.when(b1 > b0)
        def _():
            lo = (b0 // 8) * 8
            nch = (b1 - lo + K - 1) // K

            def _chunk(i, carry2):
                pos = lo + i * K
                pltpu.sync_copy(ids_hbm.at[pl.ds(pos, K)], ids_v)
                pltpu.sync_copy(srcs_hbm.at[pl.ds(pos, K)], src_v)
                pltpu.sync_copy(dsts_hbm.at[pl.ds(pos, K)],
                                dst_v.at[pl.ds(0, K)])
                cp1 = pltpu.async_copy(h_hbm.at[src_v], msg_v, sem1)
                cp2 = pltpu.async_copy(ea_hbm.at[ids_v], ea_v, sem2)
                cp1.wait()
                cp2.wait()

                def _row(r, rc):
                    p = pos + r
                    f = jnp.where((p >= b0) & (p < b1), 1.0, 0.0)
                    loc = jnp.clip(dst_v[pl.ds(r, 16)][0] - base, 0, NPW - 1)
                    for j in range(H // 16):
                        a = msg_v[r, pl.ds(j * 16, 16)]
                        b = ea_v[r, pl.ds(j * 16, 16)]
                        part_v[loc, pl.ds(j * 16, 16)] = (
                            part_v[loc, pl.ds(j * 16, 16)]
                            + jnp.maximum(a + b, 0.0) * f)
                    return rc

                lax.fori_loop(0, K, _row, 0)
                return carry2

            lax.fori_loop(0, nch, _chunk, 0)

            def _mrow(r, carry3):
                for j in range(H // 16):
                    acc_v[r, pl.ds(j * 16, 16)] = (
                        acc_v[r, pl.ds(j * 16, 16)]
                        + part_v[r, pl.ds(j * 16, 16)])
                    part_v[r, pl.ds(j * 16, 16)] = jnp.zeros((16,),
                                                             jnp.float32)
                return carry3

            lax.fori_loop(0, NPW, _mrow, 0)

        return carry

    lax.fori_loop(0, _NSEG, _segment, 0)

    pltpu.sync_copy(acc_v, out_hbm.at[pl.ds(base, NPW)])


_NBLK = 10
_BLK = N // _NBLK
_EBLK = 4000
_ENBLK = E // _EBLK


_PREC = None


def _enc_body(x_ref, w_ref, b_ref, o_ref):
    o_ref[...] = (jnp.dot(x_ref[...], w_ref[...], precision=_PREC,
                          preferred_element_type=jnp.float32) + b_ref[...])


def _encode_nodes(x, w, b):
    return pl.pallas_call(
        _enc_body,
        grid=(_NBLK,),
        in_specs=[
            pl.BlockSpec((_BLK, DF), lambda i: (i, 0)),
            pl.BlockSpec((DF, H), lambda i: (0, 0)),
            pl.BlockSpec((1, H), lambda i: (0, 0)),
        ],
        out_specs=pl.BlockSpec((_BLK, H), lambda i: (i, 0)),
        out_shape=jax.ShapeDtypeStruct((N, H), jnp.float32),
    )(x, w, b)


def _encode_edges(ea, w, b):
    return pl.pallas_call(
        _enc_body,
        grid=(_ENBLK,),
        in_specs=[
            pl.BlockSpec((_EBLK, ED), lambda i: (i, 0)),
            pl.BlockSpec((ED, H), lambda i: (0, 0)),
            pl.BlockSpec((1, H), lambda i: (0, 0)),
        ],
        out_specs=pl.BlockSpec((_EBLK, H), lambda i: (i, 0)),
        out_shape=jax.ShapeDtypeStruct((E, H), jnp.float32),
    )(ea, w, b)


def _mlp_body(h_ref, a0_ref, w1_ref, b1_ref, w2_ref, b2_ref,
              z_ref, st_ref, s1):
    i = pl.program_id(0)
    t = h_ref[...] + a0_ref[...]
    u = jnp.maximum(jnp.dot(t, w1_ref[...], precision=_PREC,
                            preferred_element_type=jnp.float32) + b1_ref[...],
                    0.0)
    z = (jnp.dot(u, w2_ref[...], precision=_PREC,
                 preferred_element_type=jnp.float32) + b2_ref[...])
    z_ref[...] = z

    @pl.when(i == 0)
    def _():
        s1[...] = jnp.zeros_like(s1)

    s1[...] += jnp.sum(z, 0, keepdims=True)

    @pl.when(i == pl.num_programs(0) - 1)
    def _():
        st_ref[...] = s1[...]


def _mlp(h, a0, w1, b1, w2, b2):
    return pl.pallas_call(
        _mlp_body,
        grid=(_NBLK,),
        in_specs=[
            pl.BlockSpec((_BLK, H), lambda i: (i, 0)),
            pl.BlockSpec((_BLK, H), lambda i: (i, 0)),
            pl.BlockSpec((H, H), lambda i: (0, 0)),
            pl.BlockSpec((1, H), lambda i: (0, 0)),
            pl.BlockSpec((H, H), lambda i: (0, 0)),
            pl.BlockSpec((1, H), lambda i: (0, 0)),
        ],
        out_specs=[
            pl.BlockSpec((_BLK, H), lambda i: (i, 0)),
            pl.BlockSpec((1, H), lambda i: (0, 0)),
        ],
        out_shape=[
            jax.ShapeDtypeStruct((N, H), jnp.float32),
            jax.ShapeDtypeStruct((1, H), jnp.float32),
        ],
        scratch_shapes=[
            pltpu.VMEM((1, H), jnp.float32),
        ],
    )(h, a0, w1, b1, w2, b2)


def _var_body(z_ref, s1_ref, sv_ref, sv):
    i = pl.program_id(0)
    mu = s1_ref[...] * (1.0 / N)

    @pl.when(i == 0)
    def _():
        sv[...] = jnp.zeros_like(sv)

    d = z_ref[...] - mu
    sv[...] += jnp.sum(d * d, 0, keepdims=True)

    @pl.when(i == pl.num_programs(0) - 1)
    def _():
        sv_ref[...] = sv[...]


def _var(z, s1):
    return pl.pallas_call(
        _var_body,
        grid=(_NBLK,),
        in_specs=[
            pl.BlockSpec((_BLK, H), lambda i: (i, 0)),
            pl.BlockSpec((1, H), lambda i: (0, 0)),
        ],
        out_specs=pl.BlockSpec((1, H), lambda i: (0, 0)),
        out_shape=jax.ShapeDtypeStruct((1, H), jnp.float32),
        scratch_shapes=[pltpu.VMEM((1, H), jnp.float32)],
    )(z, s1)


def _bnres_body(z_ref, s1_ref, sv_ref, g_ref, be_ref, r_ref, o_ref):
    mu = s1_ref[...] * (1.0 / N)
    var = sv_ref[...] * (1.0 / N)
    scale = g_ref[...] / jnp.sqrt(var + 1e-5)
    shift = be_ref[...] - mu * scale
    o_ref[...] = jnp.maximum(z_ref[...] * scale + shift, 0.0) + r_ref[...]


def _bnres(z, s1, sv, g, be, res):
    return pl.pallas_call(
        _bnres_body,
        grid=(_NBLK,),
        in_specs=[
            pl.BlockSpec((_BLK, H), lambda i: (i, 0)),
            pl.BlockSpec((1, H), lambda i: (0, 0)),
            pl.BlockSpec((1, H), lambda i: (0, 0)),
            pl.BlockSpec((1, H), lambda i: (0, 0)),
            pl.BlockSpec((1, H), lambda i: (0, 0)),
            pl.BlockSpec((_BLK, H), lambda i: (i, 0)),
        ],
        out_specs=pl.BlockSpec((_BLK, H), lambda i: (i, 0)),
        out_shape=jax.ShapeDtypeStruct((N, H), jnp.float32),
    )(z, s1, sv, g, be, res)


def _pool_head_body(bf_ref, h_ref, w1_ref, b1_ref, g_ref, be_ref,
                    w2_ref, b2_ref, o_ref, pool_s, cnt_s):
    i = pl.program_id(0)

    @pl.when(i == 0)
    def _():
        pool_s[...] = jnp.zeros_like(pool_s)
        cnt_s[...] = jnp.zeros_like(cnt_s)

    b = bf_ref[...]
    iota = lax.broadcasted_iota(jnp.int32, (_BLK, NG), 1).astype(jnp.float32)
    onehot = (b == iota).astype(jnp.float32)
    dn = (((0,), (0,)), ((), ()))
    pool_s[...] += lax.dot_general(onehot, h_ref[...], dn, precision=_PREC,
                                   preferred_element_type=jnp.float32)
    cnt_s[...] += lax.dot_general(onehot, jnp.ones((_BLK, H), jnp.float32),
                                  dn, precision=_PREC,
                                  preferred_element_type=jnp.float32)

    @pl.when(i == pl.num_programs(0) - 1)
    def _():
        pooled = pool_s[...] / jnp.maximum(cnt_s[...], 1.0)
        y = (jnp.dot(pooled, w1_ref[...], precision=_PREC,
                     preferred_element_type=jnp.float32) + b1_ref[...])
        mu = jnp.mean(y, 0, keepdims=True)
        var = jnp.mean((y - mu) * (y - mu), 0, keepdims=True)
        y = g_ref[...] * (y - mu) / jnp.sqrt(var + 1e-5) + be_ref[...]
        y = jnp.maximum(y, 0.0)
        o_ref[...] = (jnp.dot(y, w2_ref[...], precision=_PREC,
                              preferred_element_type=jnp.float32) + b2_ref[...])


def _pool_head(bf, h, w1, b1, g, be, w2p, b2p):
    return pl.pallas_call(
        _pool_head_body,
        grid=(_NBLK,),
        in_specs=[
            pl.BlockSpec((_BLK, 1), lambda i: (i, 0)),
            pl.BlockSpec((_BLK, H), lambda i: (i, 0)),
            pl.BlockSpec((H, H), lambda i: (0, 0)),
            pl.BlockSpec((1, H), lambda i: (0, 0)),
            pl.BlockSpec((1, H), lambda i: (0, 0)),
            pl.BlockSpec((1, H), lambda i: (0, 0)),
            pl.BlockSpec((H, H), lambda i: (0, 0)),
            pl.BlockSpec((1, H), lambda i: (0, 0)),
        ],
        out_specs=pl.BlockSpec((NG, H), lambda i: (0, 0)),
        out_shape=jax.ShapeDtypeStruct((NG, H), jnp.float32),
        scratch_shapes=[
            pltpu.VMEM((NG, H), jnp.float32),
            pltpu.VMEM((NG, H), jnp.float32),
        ],
    )(bf, h, w1, b1, g, be, w2p, b2p)


def kernel(x, edge_index, edge_attr, batch, params):
    src = edge_index[0].astype(jnp.int32)
    dst = edge_index[1].astype(jnp.int32)
    bf = batch.astype(jnp.float32).reshape(N, 1)

    perm = jnp.argsort(dst, stable=True).astype(jnp.int32)
    dst_s = dst[perm]
    src_s = src[perm]
    bnd = jnp.searchsorted(
        dst_s, jnp.arange(0, N_PAD + 1, NPW, dtype=jnp.int32)).astype(jnp.int32)
    b0c = bnd[:-1, None]
    b1c = bnd[1:, None]
    plist = jnp.asarray(_P_LIST, jnp.int32)[None, :]
    segs = jnp.concatenate(
        [b0c, jnp.clip(plist, b0c, b1c),
         jnp.broadcast_to(b1c, (NW, _SEGW - len(_P_LIST) - 1))],
        axis=1).astype(jnp.int32)
    ids_pad = jnp.pad(perm, (0, PADE))
    src_pad = jnp.pad(src_s, (0, PADE))
    dst_pad = jnp.pad(dst_s, (0, PADE))

    h = _encode_nodes(x, params["x_enc"]["w"],
                      params["x_enc"]["b"].reshape(1, H))
    ea = _encode_edges(edge_attr, params["edge_enc"]["w"],
                       params["edge_enc"]["b"].reshape(1, H))

    for i in range(4):
        cv = params["convs"][i]
        aggr = _sc_aggregate(h, ea, ids_pad, src_pad, dst_pad, segs)[:N]
        z = h + aggr
        z = (jax.nn.relu(z @ cv["lin1"]["w"] + cv["lin1"]["b"])
             @ cv["lin2"]["w"] + cv["lin2"]["b"])
        mu = jnp.mean(z, 0)
        var = jnp.var(z, 0)
        z = (params["bns"][i]["gamma"] * (z - mu) / jnp.sqrt(var + 1e-5)
             + params["bns"][i]["beta"])
        h = jax.nn.relu(z) + h

    counts = jax.ops.segment_sum(jnp.ones((N, 1), jnp.float32),
                                 batch, num_segments=NG)
    pooled = (jax.ops.segment_sum(h, batch, num_segments=NG)
              / jnp.maximum(counts, 1.0))
    out = pooled @ params["lin1"]["w"] + params["lin1"]["b"]
    mu = jnp.mean(out, 0)
    var = jnp.var(out, 0)
    out = (params["bn_final"]["gamma"] * (out - mu)
           / jnp.sqrt(var + 1e-5) + params["bn_final"]["beta"])
    out = jax.nn.relu(out)
    return out @ params["lin2"]["w"] + params["lin2"]["b"]
